# baseline (speedup 1.0000x reference)
import jax
import jax.numpy as jnp
from jax import lax
from jax.experimental import pallas as pl
from jax.experimental.pallas import tpu as pltpu
from jax.experimental.pallas import tpu_sc as plsc

BATCH = 16384
HIST = 50
EMBED_DIM = 64
B = BATCH * HIST
NC, NS = 2, 16
NW = NC * NS
BPW = BATCH // NW
NBB = BPW // 128
HALF = 256
NUNITS = HIST * 2


def _lookup_kernel(dataT_hbm, table_hbm, out_hbm, idx_v, rows_v, st_v,
                   sem_g0, sem_g1, sem_o0, sem_o1):
    wid = lax.axis_index("s") * NC + lax.axis_index("c")
    boff = wid * BPW
    sem_g = (sem_g0, sem_g1)
    sem_o = (sem_o0, sem_o1)

    pltpu.sync_copy(dataT_hbm.at[:, pl.ds(boff, BPW)], idx_v)

    iota = lax.iota(jnp.int32, 16)

    def start_gather(u, p):
        h = u >> 1
        idx_ref = idx_v.at[h].at[pl.ds((u & 1) * HALF, HALF)]
        pltpu.async_copy(table_hbm.at[idx_ref], rows_v.at[p], sem_g[p])

    def wait_gather(u, p):
        h = u >> 1
        idx_ref = idx_v.at[h].at[pl.ds((u & 1) * HALF, HALF)]
        pltpu.make_async_copy(table_hbm.at[idx_ref], rows_v.at[p],
                              sem_g[p]).wait()

    def start_outs(u, p):
        h = u >> 1
        bb = wid * NBB + (u & 1) * 2
        for cb in range(8):
            for bbl in range(2):
                pltpu.async_copy(
                    st_v.at[p].at[pl.ds(cb * 16 + bbl * 8, 8)]
                    .at[:, pl.ds(0, 128)],
                    out_hbm.at[h].at[cb].at[bb + bbl], sem_o[p])

    def wait_outs(u, p):
        h = u >> 1
        bb = wid * NBB + (u & 1) * 2
        for cb in range(8):
            for bbl in range(2):
                pltpu.make_async_copy(
                    st_v.at[p].at[pl.ds(cb * 16 + bbl * 8, 8)]
                    .at[:, pl.ds(0, 128)],
                    out_hbm.at[h].at[cb].at[bb + bbl], sem_o[p]).wait()

    rowv = [((c0 + iota) // 8) * 16 + (c0 + iota) % 8
            for c0 in (0, 16, 32, 48)]

    def transpose(p):
        rows = rows_v.at[p]
        st2 = st_v.at[p]

        def body(t, carry):
            for k in range(4):
                r = t * 4 + k
                bbl8 = jnp.full((16,), (r >> 7) * 8, jnp.int32)
                blv = jnp.full((16,), r & 127, jnp.int32)
                for g in range(4):
                    vals = rows[r, pl.ds(g * 16, 16)]
                    plsc.store_scatter(st2, [rowv[g] + bbl8, blv], vals)
            return carry

        lax.fori_loop(0, HALF // 4, body, 0)

    start_gather(0, 0)

    def body(j, carry):
        for p in range(2):
            u = 2 * j + p

            @pl.when(u + 1 < NUNITS)
            def _():
                start_gather(u + 1, 1 - p)

            wait_gather(u, p)

            @pl.when(u >= 2)
            def _():
                wait_outs(u - 2, p)

            transpose(p)
            start_outs(u, p)
        return carry

    lax.fori_loop(0, NUNITS // 2, body, 0)
    wait_outs(NUNITS - 2, 0)
    wait_outs(NUNITS - 1, 1)


def kernel(data, embedding_table):
    dataT = jnp.transpose(data)
    mesh = plsc.VectorSubcoreMesh(core_axis_name="c", subcore_axis_name="s")
    out5 = pl.kernel(
        _lookup_kernel,
        out_type=jax.ShapeDtypeStruct((HIST, 8, 128, 8, 128), jnp.float32),
        mesh=mesh,
        scratch_types=[
            pltpu.VMEM((HIST, BPW), jnp.int32),
            pltpu.VMEM((2, HALF, EMBED_DIM), jnp.float32),
            pltpu.VMEM((2, 128, 129), jnp.float32),
            pltpu.SemaphoreType.DMA,
            pltpu.SemaphoreType.DMA,
            pltpu.SemaphoreType.DMA,
            pltpu.SemaphoreType.DMA,
        ],
        compiler_params=pltpu.CompilerParams(use_tc_tiling_on_sc=False,
                                             needs_layout_passes=False),
    )(dataT, embedding_table)
    return out5.transpose(2, 4, 0, 1, 3).reshape(BATCH, HIST, EMBED_DIM)


# speedup vs baseline: 1.0020x; 1.0020x over previous
import jax
import jax.numpy as jnp
from jax import lax
from jax.experimental import pallas as pl
from jax.experimental.pallas import tpu as pltpu
from jax.experimental.pallas import tpu_sc as plsc

BATCH = 16384
HIST = 50
EMBED_DIM = 64
B = BATCH * HIST
NC, NS = 2, 16
NW = NC * NS
BPW = BATCH // NW
NBB = BPW // 128
HALF = 256
NUNITS = HIST * 2


def _lookup_kernel(dataT_hbm, table_hbm, out_hbm, idx_v, rows_v, st_v,
                   sem_g0, sem_g1, sem_o0, sem_o1):
    wid = lax.axis_index("s") * NC + lax.axis_index("c")
    boff = wid * BPW
    sem_g = (sem_g0, sem_g1)
    sem_o = (sem_o0, sem_o1)

    pltpu.sync_copy(dataT_hbm.at[:, pl.ds(boff, BPW)], idx_v)

    iota = lax.iota(jnp.int32, 16)

    def start_gather(u, p):
        h = u >> 1
        idx_ref = idx_v.at[h].at[pl.ds((u & 1) * HALF, HALF)]
        pltpu.async_copy(table_hbm.at[idx_ref], rows_v.at[p], sem_g[p])

    def wait_gather(u, p):
        h = u >> 1
        idx_ref = idx_v.at[h].at[pl.ds((u & 1) * HALF, HALF)]
        pltpu.make_async_copy(table_hbm.at[idx_ref], rows_v.at[p],
                              sem_g[p]).wait()

    def start_outs(u, p):
        h = u >> 1
        bb = wid * NBB + (u & 1) * 2
        for cb in range(8):
            for bbl in range(2):
                pltpu.async_copy(
                    st_v.at[p].at[pl.ds(cb * 16 + bbl * 8, 8)]
                    .at[:, pl.ds(0, 128)],
                    out_hbm.at[h].at[cb].at[bb + bbl], sem_o[p])

    def wait_outs(u, p):
        h = u >> 1
        bb = wid * NBB + (u & 1) * 2
        for cb in range(8):
            for bbl in range(2):
                pltpu.make_async_copy(
                    st_v.at[p].at[pl.ds(cb * 16 + bbl * 8, 8)]
                    .at[:, pl.ds(0, 128)],
                    out_hbm.at[h].at[cb].at[bb + bbl], sem_o[p]).wait()

    rowv = [((c0 + iota) // 8) * 16 + (c0 + iota) % 8
            for c0 in (0, 16, 32, 48)]

    def transpose(p):
        rows = rows_v.at[p]
        st2 = st_v.at[p]

        def body(t, carry):
            for k in range(8):
                r = t * 8 + k
                bbl8 = jnp.full((16,), (r >> 7) * 8, jnp.int32)
                blv = jnp.full((16,), r & 127, jnp.int32)
                for g in range(4):
                    vals = rows[r, pl.ds(g * 16, 16)]
                    plsc.store_scatter(st2, [rowv[g] + bbl8, blv], vals)
            return carry

        lax.fori_loop(0, HALF // 8, body, 0)

    start_gather(0, 0)

    def body(j, carry):
        for p in range(2):
            u = 2 * j + p

            @pl.when(u + 1 < NUNITS)
            def _():
                start_gather(u + 1, 1 - p)

            wait_gather(u, p)

            @pl.when(u >= 2)
            def _():
                wait_outs(u - 2, p)

            transpose(p)
            start_outs(u, p)
        return carry

    lax.fori_loop(0, NUNITS // 2, body, 0)
    wait_outs(NUNITS - 2, 0)
    wait_outs(NUNITS - 1, 1)


def kernel(data, embedding_table):
    dataT = jnp.transpose(data)
    mesh = plsc.VectorSubcoreMesh(core_axis_name="c", subcore_axis_name="s")
    out5 = pl.kernel(
        _lookup_kernel,
        out_type=jax.ShapeDtypeStruct((HIST, 8, 128, 8, 128), jnp.float32),
        mesh=mesh,
        scratch_types=[
            pltpu.VMEM((HIST, BPW), jnp.int32),
            pltpu.VMEM((2, HALF, EMBED_DIM), jnp.float32),
            pltpu.VMEM((2, 128, 129), jnp.float32),
            pltpu.SemaphoreType.DMA,
            pltpu.SemaphoreType.DMA,
            pltpu.SemaphoreType.DMA,
            pltpu.SemaphoreType.DMA,
        ],
        compiler_params=pltpu.CompilerParams(use_tc_tiling_on_sc=False,
                                             needs_layout_passes=False),
    )(dataT, embedding_table)
    return out5.transpose(2, 4, 0, 1, 3).reshape(BATCH, HIST, EMBED_DIM)
